# speedup vs baseline: 1.7678x; 1.7678x over previous
import functools

import jax
import jax.numpy as jnp
from jax import lax
from jax.experimental import pallas as pl
from jax.experimental.pallas import tpu as pltpu
from jax.experimental.pallas import tpu_sc as plsc

_BATCH = 16384
_DIM = 1024
_NC = 2
_NS = 16
_NW = _NC * _NS
_BPW = _BATCH // _NW
_CHUNK = 32
_NCHUNK = _BPW // _CHUNK


def _make_sc_gather():
    mesh = plsc.VectorSubcoreMesh(core_axis_name="c", subcore_axis_name="s")

    @functools.partial(
        pl.kernel,
        mesh=mesh,
        out_type=jax.ShapeDtypeStruct((_BATCH, _DIM), jnp.float32),
        scratch_types=[
            pltpu.VMEM((_NCHUNK, _CHUNK), jnp.int32),
            pltpu.VMEM((_CHUNK, _DIM), jnp.float32),
            pltpu.SemaphoreType.DMA,
        ],
    )
    def body(pos_hbm, table_hbm, out_hbm, idx_v, rows_v, sem):
        wid = lax.axis_index("s") * _NC + lax.axis_index("c")
        base = wid * _BPW
        pltpu.sync_copy(pos_hbm.at[wid], idx_v)
        for c in range(_NCHUNK):
            pltpu.async_copy(table_hbm.at[idx_v.at[c]], rows_v, sem).wait()
            pltpu.sync_copy(rows_v, out_hbm.at[pl.ds(base + c * _CHUNK, _CHUNK)])

    return body


_sc_gather = _make_sc_gather()


@jax.jit
def kernel(pos, table):
    pos3 = pos.astype(jnp.int32).reshape(_NW, _NCHUNK, _CHUNK)
    return _sc_gather(pos3, table)


# speedup vs baseline: 2.0329x; 1.1500x over previous
import functools

import jax
import jax.numpy as jnp
from jax import lax
from jax.experimental import pallas as pl
from jax.experimental.pallas import tpu as pltpu
from jax.experimental.pallas import tpu_sc as plsc

_BATCH = 16384
_DIM = 1024
_NC = 2
_NS = 16
_NW = _NC * _NS
_BPW = _BATCH // _NW
_CHUNK = 32
_NCHUNK = _BPW // _CHUNK
_NBUF = 3
_AHEAD = 2


def _make_sc_gather():
    mesh = plsc.VectorSubcoreMesh(core_axis_name="c", subcore_axis_name="s")

    @functools.partial(
        pl.kernel,
        mesh=mesh,
        out_type=jax.ShapeDtypeStruct((_BATCH, _DIM), jnp.float32),
        scratch_types=[
            pltpu.VMEM((_NCHUNK, _CHUNK), jnp.int32),
            pltpu.VMEM((_NBUF, _CHUNK, _DIM), jnp.float32),
            *([pltpu.SemaphoreType.DMA] * _NBUF),
            *([pltpu.SemaphoreType.DMA] * _NBUF),
        ],
    )
    def body(pos_hbm, table_hbm, out_hbm, idx_v, rows_v, *sems):
        gsem = sems[:_NBUF]
        ssem = sems[_NBUF:]
        wid = lax.axis_index("s") * _NC + lax.axis_index("c")
        base = wid * _BPW
        pltpu.sync_copy(pos_hbm.at[wid], idx_v)
        gathers = [None] * _NCHUNK
        stores = [None] * _NCHUNK
        for t in range(_NCHUNK):
            b = t % _NBUF
            if t >= _NBUF:
                stores[t - _NBUF].wait()
            gathers[t] = pltpu.async_copy(
                table_hbm.at[idx_v.at[t]], rows_v.at[b], gsem[b])
            d = t - (_AHEAD - 1)
            if d >= 0:
                gathers[d].wait()
                stores[d] = pltpu.async_copy(
                    rows_v.at[d % _NBUF],
                    out_hbm.at[pl.ds(base + d * _CHUNK, _CHUNK)],
                    ssem[d % _NBUF])
        for d in range(_NCHUNK - (_AHEAD - 1), _NCHUNK):
            gathers[d].wait()
            stores[d] = pltpu.async_copy(
                rows_v.at[d % _NBUF],
                out_hbm.at[pl.ds(base + d * _CHUNK, _CHUNK)],
                ssem[d % _NBUF])
        for d in range(_NCHUNK - _NBUF, _NCHUNK):
            stores[d].wait()

    return body


_sc_gather = _make_sc_gather()


@jax.jit
def kernel(pos, table):
    pos3 = pos.astype(jnp.int32).reshape(_NW, _NCHUNK, _CHUNK)
    return _sc_gather(pos3, table)


# speedup vs baseline: 2.0743x; 1.0204x over previous
import functools

import jax
import jax.numpy as jnp
from jax import lax
from jax.experimental import pallas as pl
from jax.experimental.pallas import tpu as pltpu
from jax.experimental.pallas import tpu_sc as plsc

_BATCH = 16384
_DIM = 1024
_NC = 2
_NS = 16
_NW = _NC * _NS
_BPW = _BATCH // _NW
_CHUNK = 32
_NCHUNK = _BPW // _CHUNK
_NBUF = 3
_AHEAD = 3


def _make_sc_gather():
    mesh = plsc.VectorSubcoreMesh(core_axis_name="c", subcore_axis_name="s")

    @functools.partial(
        pl.kernel,
        mesh=mesh,
        out_type=jax.ShapeDtypeStruct((_BATCH, _DIM), jnp.float32),
        scratch_types=[
            pltpu.VMEM((_NCHUNK, _CHUNK), jnp.int32),
            pltpu.VMEM((_NBUF, _CHUNK, _DIM), jnp.float32),
            *([pltpu.SemaphoreType.DMA] * _NBUF),
            *([pltpu.SemaphoreType.DMA] * _NBUF),
        ],
    )
    def body(pos_hbm, table_hbm, out_hbm, idx_v, rows_v, *sems):
        gsem = sems[:_NBUF]
        ssem = sems[_NBUF:]
        wid = lax.axis_index("s") * _NC + lax.axis_index("c")
        base = wid * _BPW
        pltpu.sync_copy(pos_hbm.at[wid], idx_v)
        gathers = [None] * _NCHUNK
        stores = [None] * _NCHUNK
        for t in range(_NCHUNK):
            b = t % _NBUF
            if t >= _NBUF:
                stores[t - _NBUF].wait()
            gathers[t] = pltpu.async_copy(
                table_hbm.at[idx_v.at[t]], rows_v.at[b], gsem[b])
            d = t - (_AHEAD - 1)
            if d >= 0:
                gathers[d].wait()
                stores[d] = pltpu.async_copy(
                    rows_v.at[d % _NBUF],
                    out_hbm.at[pl.ds(base + d * _CHUNK, _CHUNK)],
                    ssem[d % _NBUF])
        for d in range(_NCHUNK - (_AHEAD - 1), _NCHUNK):
            gathers[d].wait()
            stores[d] = pltpu.async_copy(
                rows_v.at[d % _NBUF],
                out_hbm.at[pl.ds(base + d * _CHUNK, _CHUNK)],
                ssem[d % _NBUF])
        for d in range(_NCHUNK - _NBUF, _NCHUNK):
            stores[d].wait()

    return body


_sc_gather = _make_sc_gather()


@jax.jit
def kernel(pos, table):
    pos3 = pos.astype(jnp.int32).reshape(_NW, _NCHUNK, _CHUNK)
    return _sc_gather(pos3, table)


# speedup vs baseline: 2.0864x; 1.0058x over previous
import functools

import jax
import jax.numpy as jnp
from jax import lax
from jax.experimental import pallas as pl
from jax.experimental.pallas import tpu as pltpu
from jax.experimental.pallas import tpu_sc as plsc

_BATCH = 16384
_DIM = 1024
_NC = 2
_NS = 16
_NW = _NC * _NS
_BPW = _BATCH // _NW
_CHUNK = 32
_NCHUNK = _BPW // _CHUNK
_NBUF = 3
_AHEAD = 3


def _make_sc_gather():
    mesh = plsc.VectorSubcoreMesh(core_axis_name="c", subcore_axis_name="s")

    @functools.partial(
        pl.kernel,
        mesh=mesh,
        out_type=jax.ShapeDtypeStruct((_BATCH, _DIM), jnp.float32),
        scratch_types=[
            pltpu.VMEM((_BPW,), jnp.int32),
            pltpu.VMEM((_NBUF, _CHUNK, _DIM), jnp.float32),
            *([pltpu.SemaphoreType.DMA] * _NBUF),
            *([pltpu.SemaphoreType.DMA] * _NBUF),
        ],
    )
    def body(pos_hbm, table_hbm, out_hbm, idx_v, rows_v, *sems):
        gsem = sems[:_NBUF]
        ssem = sems[_NBUF:]
        wid = lax.axis_index("s") * _NC + lax.axis_index("c")
        base = wid * _BPW
        pltpu.sync_copy(pos_hbm.at[pl.ds(base, _BPW)], idx_v)
        gathers = [None] * _NCHUNK
        stores = [None] * _NCHUNK
        for t in range(_NCHUNK):
            b = t % _NBUF
            if t >= _NBUF:
                stores[t - _NBUF].wait()
            gathers[t] = pltpu.async_copy(
                table_hbm.at[idx_v.at[pl.ds(t * _CHUNK, _CHUNK)]],
                rows_v.at[b], gsem[b])
            d = t - (_AHEAD - 1)
            if d >= 0:
                gathers[d].wait()
                stores[d] = pltpu.async_copy(
                    rows_v.at[d % _NBUF],
                    out_hbm.at[pl.ds(base + d * _CHUNK, _CHUNK)],
                    ssem[d % _NBUF])
        for d in range(_NCHUNK - (_AHEAD - 1), _NCHUNK):
            gathers[d].wait()
            stores[d] = pltpu.async_copy(
                rows_v.at[d % _NBUF],
                out_hbm.at[pl.ds(base + d * _CHUNK, _CHUNK)],
                ssem[d % _NBUF])
        for d in range(_NCHUNK - _NBUF, _NCHUNK):
            stores[d].wait()

    return body


_sc_gather = _make_sc_gather()


@jax.jit
def kernel(pos, table):
    return _sc_gather(pos.astype(jnp.int32), table)
